# baseline (speedup 1.0000x reference)
import functools

import jax
import jax.numpy as jnp
from jax import lax
from jax.experimental import pallas as pl
from jax.experimental.pallas import tpu as pltpu
from jax.experimental.pallas import tpu_sc as plsc

N = 10000
E = 320000
D = 128
HALF = N // 2

NC = 2
NS = 16
NW = NC * NS
CHW = 80

EPW = E // NW
CH1 = EPW // CHW
EPT = E // NS
CH2 = EPT // CHW

TRASH = HALF
ACC_ROWS = HALF + 8
RPT = 320
RPT_LAST = HALF - RPT * (NS - 1)

DEG_RPT = 640
DEG_N = NS * DEG_RPT

_mesh = plsc.VectorSubcoreMesh(core_axis_name="c", subcore_axis_name="s")


@functools.partial(
    pl.kernel,
    out_type=jax.ShapeDtypeStruct((NC * DEG_N,), jnp.float32),
    mesh=_mesh,
    scratch_types=[
        pltpu.VMEM((CH1, CHW), jnp.int32),
        pltpu.VMEM((CHW,), jnp.float32),
        pltpu.VMEM((DEG_RPT,), jnp.float32),
        pltpu.VMEM_SHARED((DEG_N,), jnp.float32),
    ],
)
def _deg_kernel(col_hbm, deg_out, col_v, ones_v, zero_v, deg_acc):
    c = lax.axis_index("c")
    s = lax.axis_index("s")
    w = s * NC + c

    pltpu.sync_copy(col_hbm.at[w], col_v)
    for i in range(CHW // 16):
        ones_v[pl.ds(i * 16, 16)] = jnp.ones((16,), jnp.float32)
    for i in range(DEG_RPT // 16):
        zero_v[pl.ds(i * 16, 16)] = jnp.zeros((16,), jnp.float32)
    pltpu.sync_copy(zero_v, deg_acc.at[pl.ds(s * DEG_RPT, DEG_RPT)])
    plsc.subcore_barrier()

    def body(j, _):
        pltpu.sync_copy(ones_v, deg_acc.at[col_v.at[j]], add=True)
        return 0

    lax.fori_loop(0, CH1, body, 0)
    plsc.subcore_barrier()
    pltpu.sync_copy(deg_acc.at[pl.ds(s * DEG_RPT, DEG_RPT)],
                    deg_out.at[pl.ds(c * DEG_N + s * DEG_RPT, DEG_RPT)])


CHS = 64
NG = 10
GE = EPT // NG
NCH = 316
CMP = (NCH + 2) * CHS


@functools.partial(
    pl.kernel,
    out_type=jax.ShapeDtypeStruct((NC, HALF, D), jnp.float32),
    mesh=_mesh,
    compiler_params=pltpu.CompilerParams(needs_layout_passes=False),
    scratch_types=[
        pltpu.VMEM((GE,), jnp.int32),
        pltpu.VMEM((GE,), jnp.int32),
        pltpu.VMEM((CMP,), jnp.int32),
        pltpu.VMEM((CMP,), jnp.int32),
        pltpu.VMEM((4, CHS), jnp.int32),
        pltpu.VMEM((CHS, D), jnp.float32),
        pltpu.VMEM((CHS, D), jnp.float32),
        pltpu.VMEM((CHS, D), jnp.float32),
        pltpu.VMEM((CHS, D), jnp.float32),
        pltpu.VMEM((8, D), jnp.float32),
        pltpu.VMEM_SHARED((ACC_ROWS, D), jnp.float32),
        pltpu.SemaphoreType.DMA,
        pltpu.SemaphoreType.DMA,
        pltpu.SemaphoreType.DMA,
        pltpu.SemaphoreType.DMA,
        pltpu.SemaphoreType.DMA,
        pltpu.SemaphoreType.DMA,
        pltpu.SemaphoreType.DMA,
        pltpu.SemaphoreType.DMA,
    ],
)
def _scatter_kernel(row_hbm, col_hbm, y_hbm, acc_out,
                    row_v, col_v, rflat, cflat, cidx,
                    buf0, buf1, buf2, buf3, zbuf, acc,
                    gs0, gs1, gs2, gs3, ss0, ss1, ss2, ss3):
    c = lax.axis_index("c")
    s = lax.axis_index("s")
    lo = c * HALF
    base = s * RPT
    bufs = [buf0, buf1, buf2, buf3]
    gsem = [gs0, gs1, gs2, gs3]
    ssem = [ss0, ss1, ss2, ss3]

    @pl.when(s < NS - 1)
    def _():
        pltpu.sync_copy(y_hbm.at[pl.ds(lo + base, RPT)],
                        acc.at[pl.ds(base, RPT)])

    @pl.when(s == NS - 1)
    def _():
        pltpu.sync_copy(y_hbm.at[pl.ds(lo + base, RPT_LAST)],
                        acc.at[pl.ds(base, RPT_LAST)])
        for i in range(8):
            for j in range(D // 16):
                zbuf[i, pl.ds(j * 16, 16)] = jnp.zeros((16,), jnp.float32)
        pltpu.sync_copy(zbuf, acc.at[pl.ds(HALF, 8)])

    def group(g, _):
        pltpu.sync_copy(row_hbm.at[s, g], row_v)
        pltpu.sync_copy(col_hbm.at[s, g], col_v)

        def localize(i, _):
            col16 = col_v[pl.ds(i * 16, 16)]
            row16 = row_v[pl.ds(i * 16, 16)]
            local = col16 - lo
            owned = (local >= 0) & (local < HALF)
            fbase = g * GE + i * 16
            cflat[pl.ds(fbase, 16)] = jnp.where(owned, local, TRASH)
            rflat[pl.ds(fbase, 16)] = row16
            return 0

        lax.fori_loop(0, GE // 16, localize, 0)
        return 0

    lax.fori_loop(0, NG, group, 0)

    for k in range((CMP - EPT) // 16):
        rflat[pl.ds(EPT + k * 16, 16)] = jnp.zeros((16,), jnp.int32)
        cflat[pl.ds(EPT + k * 16, 16)] = jnp.full((16,), TRASH, jnp.int32)

    plsc.subcore_barrier()

    def gather(j, p):
        return pltpu.make_async_copy(
            y_hbm.at[rflat.at[pl.ds(j * CHS, CHS)]], bufs[p], gsem[p])

    def scatter(p):
        return pltpu.make_async_copy(bufs[p], acc.at[cidx.at[p]], ssem[p])

    def step(j, p, wait_scatter):
        gather(j, p).wait()
        if wait_scatter:
            scatter((p + 2) % 4).wait()
        gather(j + 2, (p + 2) % 4).start()
        for k in range(CHS // 16):
            cidx[p, pl.ds(k * 16, 16)] = cflat[pl.ds(j * CHS + k * 16, 16)]
        pltpu.async_copy(bufs[p], acc.at[cidx.at[p]], ssem[p], add=True)

    gather(0, 0).start()
    gather(1, 1).start()
    step(0, 0, False)
    step(1, 1, False)
    step(2, 2, True)
    step(3, 3, True)

    def body(i, _):
        j = 4 * i
        for p in range(4):
            step(j + p, p, True)
        return 0

    lax.fori_loop(1, NCH // 4, body, 0)

    scatter(2).wait()
    scatter(3).wait()
    gather(NCH, 0).wait()
    gather(NCH + 1, 1).wait()

    plsc.subcore_barrier()

    @pl.when(s < NS - 1)
    def _():
        pltpu.sync_copy(acc.at[pl.ds(base, RPT)],
                        acc_out.at[c].at[pl.ds(base, RPT)])

    @pl.when(s == NS - 1)
    def _():
        pltpu.sync_copy(acc.at[pl.ds(base, RPT_LAST)],
                        acc_out.at[c].at[pl.ds(base, RPT_LAST)])


def _gru_body(x0_ref, wih_ref, whh_ref, bih_ref, bhh_ref, w_ref):
    x0 = x0_ref[...]
    dn = (((1,), (1,)), ((), ()))
    gi = lax.dot_general(x0, wih_ref[...], dn,
                         preferred_element_type=jnp.float32) + bih_ref[...]
    gh = lax.dot_general(x0, whh_ref[...], dn,
                         preferred_element_type=jnp.float32) + bhh_ref[...]
    r = jax.nn.sigmoid(gi[:, 0:D] + gh[:, 0:D])
    z = jax.nn.sigmoid(gi[:, D:2 * D] + gh[:, D:2 * D])
    n = jnp.tanh(gi[:, 2 * D:3 * D] + r * gh[:, 2 * D:3 * D])
    w_ref[...] = (1.0 - z) * n + z * x0


_gru = pl.pallas_call(
    _gru_body,
    out_shape=jax.ShapeDtypeStruct((D, D), jnp.float32),
)


_YBLK = 1000


def _y_body(x_ref, w_ref, degt_ref, y_ref):
    dn = (((1,), (0,)), ((), ()))
    xw = lax.dot_general(x_ref[...], w_ref[...], dn,
                         preferred_element_type=jnp.float32)
    dp = degt_ref[...]
    dinv = lax.rsqrt(dp[:, 0:1] + dp[:, 1:2] + 1.0)
    y_ref[...] = dinv * xw


_y_call = pl.pallas_call(
    _y_body,
    grid=(N // _YBLK,),
    in_specs=[
        pl.BlockSpec((_YBLK, D), lambda i: (i, 0)),
        pl.BlockSpec((D, D), lambda i: (0, 0)),
        pl.BlockSpec((_YBLK, 2), lambda i: (i, 0)),
    ],
    out_specs=pl.BlockSpec((_YBLK, D), lambda i: (i, 0)),
    out_shape=jax.ShapeDtypeStruct((N, D), jnp.float32),
)


_CBLK = 1000


def _comb_body(acc_ref, degt_ref, out_ref):
    a = acc_ref[0]
    dp = degt_ref[...]
    dinv = lax.rsqrt(dp[:, 0:1] + dp[:, 1:2] + 1.0)
    out_ref[...] = dinv * a


_comb_call = pl.pallas_call(
    _comb_body,
    grid=(N // _CBLK,),
    in_specs=[
        pl.BlockSpec((1, _CBLK, D),
                     lambda i: (i // (HALF // _CBLK), i % (HALF // _CBLK), 0)),
        pl.BlockSpec((_CBLK, 2), lambda i: (i, 0)),
    ],
    out_specs=pl.BlockSpec((_CBLK, D), lambda i: (i, 0)),
    out_shape=jax.ShapeDtypeStruct((N, D), jnp.float32),
)


def kernel(edge_index, X, initial_weight, W_ih, W_hh, b_ih, b_hh):
    row_w = edge_index[0].astype(jnp.int32).reshape(NW, CH1, CHW)
    col_w = edge_index[1].astype(jnp.int32).reshape(NW, CH1, CHW)
    row_t = edge_index[0].astype(jnp.int32).reshape(NS, NG, GE)
    col_t = edge_index[1].astype(jnp.int32).reshape(NS, NG, GE)

    W = _gru(initial_weight[0], W_ih, W_hh,
             b_ih.reshape(1, 3 * D), b_hh.reshape(1, 3 * D))

    deg_parts = _deg_kernel(col_w)
    degt = deg_parts.reshape(NC, DEG_N)[:, :N].T

    y = _y_call(X, W, degt)
    acc = _scatter_kernel(row_t, col_t, y)
    out = _comb_call(acc, degt)
    return out


# speedup vs baseline: 1.7661x; 1.7661x over previous
import functools

import jax
import jax.numpy as jnp
from jax import lax
from jax.experimental import pallas as pl
from jax.experimental.pallas import tpu as pltpu
from jax.experimental.pallas import tpu_sc as plsc

N = 10000
E = 320000
D = 128
HD = D // 2

NC = 2
NS = 16
NW = NC * NS
CHW = 80

EPW = E // NW
CH1 = EPW // CHW

TRASH = N
ACC_ROWS = N + 8
RPT = 632
RPT_LAST = N - RPT * (NS - 1)

DEG_RPT = 640
DEG_N = NS * DEG_RPT

_mesh = plsc.VectorSubcoreMesh(core_axis_name="c", subcore_axis_name="s")


@functools.partial(
    pl.kernel,
    out_type=jax.ShapeDtypeStruct((NC * DEG_N,), jnp.float32),
    mesh=_mesh,
    scratch_types=[
        pltpu.VMEM((CH1, CHW), jnp.int32),
        pltpu.VMEM((CHW,), jnp.float32),
        pltpu.VMEM((DEG_RPT,), jnp.float32),
        pltpu.VMEM_SHARED((DEG_N,), jnp.float32),
    ],
)
def _deg_kernel(col_hbm, deg_out, col_v, ones_v, zero_v, deg_acc):
    c = lax.axis_index("c")
    s = lax.axis_index("s")
    w = s * NC + c

    pltpu.sync_copy(col_hbm.at[w], col_v)
    for i in range(CHW // 16):
        ones_v[pl.ds(i * 16, 16)] = jnp.ones((16,), jnp.float32)
    for i in range(DEG_RPT // 16):
        zero_v[pl.ds(i * 16, 16)] = jnp.zeros((16,), jnp.float32)
    pltpu.sync_copy(zero_v, deg_acc.at[pl.ds(s * DEG_RPT, DEG_RPT)])
    plsc.subcore_barrier()

    def body(j, _):
        pltpu.sync_copy(ones_v, deg_acc.at[col_v.at[j]], add=True)
        return 0

    lax.fori_loop(0, CH1, body, 0)
    plsc.subcore_barrier()
    pltpu.sync_copy(deg_acc.at[pl.ds(s * DEG_RPT, DEG_RPT)],
                    deg_out.at[pl.ds(c * DEG_N + s * DEG_RPT, DEG_RPT)])


CHS = 128
NCH = 160
EPTP = (NCH + 2) * CHS


@functools.partial(
    pl.kernel,
    out_type=jax.ShapeDtypeStruct((NC, N, HD), jnp.float32),
    mesh=_mesh,
    compiler_params=pltpu.CompilerParams(needs_layout_passes=False,
                                         use_tc_tiling_on_sc=False),
    scratch_types=[
        pltpu.VMEM((NCH + 2, CHS), jnp.int32),
        pltpu.VMEM((NCH + 2, CHS), jnp.int32),
        pltpu.VMEM((CHS, HD), jnp.float32),
        pltpu.VMEM((CHS, HD), jnp.float32),
        pltpu.VMEM((CHS, HD), jnp.float32),
        pltpu.VMEM((CHS, HD), jnp.float32),
        pltpu.VMEM_SHARED((ACC_ROWS, HD), jnp.float32),
        pltpu.SemaphoreType.DMA,
        pltpu.SemaphoreType.DMA,
        pltpu.SemaphoreType.DMA,
        pltpu.SemaphoreType.DMA,
        pltpu.SemaphoreType.DMA,
        pltpu.SemaphoreType.DMA,
        pltpu.SemaphoreType.DMA,
        pltpu.SemaphoreType.DMA,
    ],
)
def _prop_kernel(row_hbm, col_hbm, yh_hbm, acc_out,
                 row_v, col_v, buf0, buf1, buf2, buf3, acc,
                 gs0, gs1, gs2, gs3, ss0, ss1, ss2, ss3):
    c = lax.axis_index("c")
    s = lax.axis_index("s")
    base = s * RPT
    bufs = [buf0, buf1, buf2, buf3]
    gsem = [gs0, gs1, gs2, gs3]
    ssem = [ss0, ss1, ss2, ss3]

    pltpu.sync_copy(row_hbm.at[s], row_v)
    pltpu.sync_copy(col_hbm.at[s], col_v)

    @pl.when(s < NS - 1)
    def _():
        pltpu.sync_copy(yh_hbm.at[c].at[pl.ds(base, RPT)],
                        acc.at[pl.ds(base, RPT)])

    @pl.when(s == NS - 1)
    def _():
        pltpu.sync_copy(yh_hbm.at[c].at[pl.ds(base, RPT_LAST)],
                        acc.at[pl.ds(base, RPT_LAST)])

    plsc.subcore_barrier()

    def gather(j, p):
        return pltpu.make_async_copy(yh_hbm.at[c].at[row_v.at[j]],
                                     bufs[p], gsem[p])

    def scatter(j, p):
        return pltpu.make_async_copy(bufs[p], acc.at[col_v.at[j]], ssem[p])

    def step(j, p, wait_scatter):
        gather(j, p).wait()
        if wait_scatter:
            scatter(j - 2, (p + 2) % 4).wait()
        gather(j + 2, (p + 2) % 4).start()
        pltpu.async_copy(bufs[p], acc.at[col_v.at[j]], ssem[p], add=True)

    gather(0, 0).start()
    gather(1, 1).start()
    step(0, 0, False)
    step(1, 1, False)
    step(2, 2, True)
    step(3, 3, True)

    def body(i, _):
        j = 4 * i
        for p in range(4):
            step(j + p, p, True)
        return 0

    lax.fori_loop(1, NCH // 4, body, 0)

    scatter(NCH - 2, 2).wait()
    scatter(NCH - 1, 3).wait()
    gather(NCH, 0).wait()
    gather(NCH + 1, 1).wait()

    plsc.subcore_barrier()

    @pl.when(s < NS - 1)
    def _():
        pltpu.sync_copy(acc.at[pl.ds(base, RPT)],
                        acc_out.at[c].at[pl.ds(base, RPT)])

    @pl.when(s == NS - 1)
    def _():
        pltpu.sync_copy(acc.at[pl.ds(base, RPT_LAST)],
                        acc_out.at[c].at[pl.ds(base, RPT_LAST)])


def _gru_body(x0_ref, wih_ref, whh_ref, bih_ref, bhh_ref, w_ref):
    x0 = x0_ref[...]
    dn = (((1,), (1,)), ((), ()))
    gi = lax.dot_general(x0, wih_ref[...], dn,
                         preferred_element_type=jnp.float32) + bih_ref[...]
    gh = lax.dot_general(x0, whh_ref[...], dn,
                         preferred_element_type=jnp.float32) + bhh_ref[...]
    r = jax.nn.sigmoid(gi[:, 0:D] + gh[:, 0:D])
    z = jax.nn.sigmoid(gi[:, D:2 * D] + gh[:, D:2 * D])
    n = jnp.tanh(gi[:, 2 * D:3 * D] + r * gh[:, 2 * D:3 * D])
    w_ref[...] = (1.0 - z) * n + z * x0


_gru = pl.pallas_call(
    _gru_body,
    out_shape=jax.ShapeDtypeStruct((D, D), jnp.float32),
)


_YBLK = 1000


def _y_body(x_ref, w_ref, degt_ref, yh_ref):
    dn = (((1,), (0,)), ((), ()))
    xw = lax.dot_general(x_ref[...], w_ref[...], dn,
                         preferred_element_type=jnp.float32)
    dp = degt_ref[...]
    dinv = lax.rsqrt(dp[:, 0:1] + dp[:, 1:2] + 1.0)
    y = dinv * xw
    yh_ref[0] = y[:, 0:HD]
    yh_ref[1] = y[:, HD:D]


_y_call = pl.pallas_call(
    _y_body,
    grid=(N // _YBLK,),
    in_specs=[
        pl.BlockSpec((_YBLK, D), lambda i: (i, 0)),
        pl.BlockSpec((D, D), lambda i: (0, 0)),
        pl.BlockSpec((_YBLK, 2), lambda i: (i, 0)),
    ],
    out_specs=pl.BlockSpec((2, _YBLK, HD), lambda i: (0, i, 0)),
    out_shape=jax.ShapeDtypeStruct((NC, N, HD), jnp.float32),
)


_CBLK = 1000


def _comb_body(acc_ref, degt_ref, out_ref):
    dp = degt_ref[...]
    dinv = lax.rsqrt(dp[:, 0:1] + dp[:, 1:2] + 1.0)
    out_ref[...] = dinv * jnp.concatenate([acc_ref[0], acc_ref[1]], axis=1)


_comb_call = pl.pallas_call(
    _comb_body,
    grid=(N // _CBLK,),
    in_specs=[
        pl.BlockSpec((2, _CBLK, HD), lambda i: (0, i, 0)),
        pl.BlockSpec((_CBLK, 2), lambda i: (i, 0)),
    ],
    out_specs=pl.BlockSpec((_CBLK, D), lambda i: (i, 0)),
    out_shape=jax.ShapeDtypeStruct((N, D), jnp.float32),
)


def kernel(edge_index, X, initial_weight, W_ih, W_hh, b_ih, b_hh):
    row32 = edge_index[0].astype(jnp.int32)
    col32 = edge_index[1].astype(jnp.int32)
    row_w = row32.reshape(NW, CH1, CHW)
    col_w = col32.reshape(NW, CH1, CHW)
    pad = EPTP - E // NS
    row_t = jnp.concatenate(
        [row32.reshape(NS, E // NS),
         jnp.zeros((NS, pad), jnp.int32)], axis=1).reshape(NS, NCH + 2, CHS)
    col_t = jnp.concatenate(
        [col32.reshape(NS, E // NS),
         jnp.full((NS, pad), TRASH, jnp.int32)], axis=1).reshape(
             NS, NCH + 2, CHS)

    W = _gru(initial_weight[0], W_ih, W_hh,
             b_ih.reshape(1, 3 * D), b_hh.reshape(1, 3 * D))

    deg_parts = _deg_kernel(col_w)
    degt = deg_parts.reshape(NC, DEG_N)[:, :N].T

    yh = _y_call(X, W, degt)
    acc = _prop_kernel(row_t, col_t, yh)
    out = _comb_call(acc, degt)
    return out


# speedup vs baseline: 2.1214x; 1.2011x over previous
import functools

import jax
import jax.numpy as jnp
from jax import lax
from jax.experimental import pallas as pl
from jax.experimental.pallas import tpu as pltpu
from jax.experimental.pallas import tpu_sc as plsc

N = 10000
E = 320000
D = 128
HD = D // 2

NC = 2
NS = 16
NW = NC * NS
CHW = 80

EPW = E // NW
CH1 = EPW // CHW

TRASH = N
ACC_ROWS = N + 8
RPT = 632
RPT_LAST = N - RPT * (NS - 1)

DEG_RPT = 640
DEG_N = NS * DEG_RPT

_mesh = plsc.VectorSubcoreMesh(core_axis_name="c", subcore_axis_name="s")


@functools.partial(
    pl.kernel,
    out_type=jax.ShapeDtypeStruct((NC * DEG_N,), jnp.float32),
    mesh=_mesh,
    scratch_types=[
        pltpu.VMEM((CH1, CHW), jnp.int32),
        pltpu.VMEM((CHW,), jnp.float32),
        pltpu.VMEM((DEG_RPT,), jnp.float32),
        pltpu.VMEM_SHARED((DEG_N,), jnp.float32),
    ],
)
def _deg_kernel(col_hbm, deg_out, col_v, ones_v, zero_v, deg_acc):
    c = lax.axis_index("c")
    s = lax.axis_index("s")
    w = s * NC + c

    pltpu.sync_copy(col_hbm.at[w], col_v)
    for i in range(CHW // 16):
        ones_v[pl.ds(i * 16, 16)] = jnp.ones((16,), jnp.float32)
    for i in range(DEG_RPT // 16):
        zero_v[pl.ds(i * 16, 16)] = jnp.zeros((16,), jnp.float32)
    pltpu.sync_copy(zero_v, deg_acc.at[pl.ds(s * DEG_RPT, DEG_RPT)])
    plsc.subcore_barrier()

    def body(j, _):
        pltpu.sync_copy(ones_v, deg_acc.at[col_v.at[j]], add=True)
        return 0

    lax.fori_loop(0, CH1, body, 0)
    plsc.subcore_barrier()
    pltpu.sync_copy(deg_acc.at[pl.ds(s * DEG_RPT, DEG_RPT)],
                    deg_out.at[pl.ds(c * DEG_N + s * DEG_RPT, DEG_RPT)])


CHS = 256
NCH = 80
EPTP = NCH * CHS


@functools.partial(
    pl.kernel,
    out_type=jax.ShapeDtypeStruct((NC, N, HD), jnp.float32),
    mesh=_mesh,
    compiler_params=pltpu.CompilerParams(needs_layout_passes=False,
                                         use_tc_tiling_on_sc=False),
    scratch_types=[
        pltpu.VMEM((NCH, CHS), jnp.int32),
        pltpu.VMEM((NCH, CHS), jnp.int32),
        pltpu.VMEM((CHS, HD), jnp.float32),
        pltpu.VMEM((CHS, HD), jnp.float32),
        pltpu.VMEM_SHARED((ACC_ROWS, HD), jnp.float32),
        pltpu.SemaphoreType.DMA,
        pltpu.SemaphoreType.DMA,
    ],
)
def _prop_kernel(row_hbm, col_hbm, yh_hbm, acc_out,
                 row_v, col_v, buf0, buf1, acc, sem0, sem1):
    c = lax.axis_index("c")
    s = lax.axis_index("s")
    base = s * RPT

    pltpu.sync_copy(row_hbm.at[s], row_v)
    pltpu.sync_copy(col_hbm.at[s], col_v)

    @pl.when(s < NS - 1)
    def _():
        pltpu.sync_copy(yh_hbm.at[c].at[pl.ds(base, RPT)],
                        acc.at[pl.ds(base, RPT)])

    @pl.when(s == NS - 1)
    def _():
        pltpu.sync_copy(yh_hbm.at[c].at[pl.ds(base, RPT_LAST)],
                        acc.at[pl.ds(base, RPT_LAST)])

    plsc.subcore_barrier()

    def gather(j, buf, sem):
        return pltpu.make_async_copy(yh_hbm.at[c].at[row_v.at[j]], buf, sem)

    gather(0, buf0, sem0).start()

    def body(i, _):
        j = 2 * i
        gather(j, buf0, sem0).wait()
        gather(j + 1, buf1, sem1).start()
        pltpu.sync_copy(buf0, acc.at[col_v.at[j]], add=True)
        gather(j + 1, buf1, sem1).wait()
        gather(j + 2, buf0, sem0).start()
        pltpu.sync_copy(buf1, acc.at[col_v.at[j + 1]], add=True)
        return 0

    lax.fori_loop(0, NCH // 2 - 1, body, 0)
    gather(NCH - 2, buf0, sem0).wait()
    gather(NCH - 1, buf1, sem1).start()
    pltpu.sync_copy(buf0, acc.at[col_v.at[NCH - 2]], add=True)
    gather(NCH - 1, buf1, sem1).wait()
    pltpu.sync_copy(buf1, acc.at[col_v.at[NCH - 1]], add=True)

    plsc.subcore_barrier()

    @pl.when(s < NS - 1)
    def _():
        pltpu.sync_copy(acc.at[pl.ds(base, RPT)],
                        acc_out.at[c].at[pl.ds(base, RPT)])

    @pl.when(s == NS - 1)
    def _():
        pltpu.sync_copy(acc.at[pl.ds(base, RPT_LAST)],
                        acc_out.at[c].at[pl.ds(base, RPT_LAST)])


def _gru_body(x0_ref, wih_ref, whh_ref, bih_ref, bhh_ref, w_ref):
    x0 = x0_ref[...]
    dn = (((1,), (1,)), ((), ()))
    gi = lax.dot_general(x0, wih_ref[...], dn,
                         preferred_element_type=jnp.float32) + bih_ref[...]
    gh = lax.dot_general(x0, whh_ref[...], dn,
                         preferred_element_type=jnp.float32) + bhh_ref[...]
    r = jax.nn.sigmoid(gi[:, 0:D] + gh[:, 0:D])
    z = jax.nn.sigmoid(gi[:, D:2 * D] + gh[:, D:2 * D])
    n = jnp.tanh(gi[:, 2 * D:3 * D] + r * gh[:, 2 * D:3 * D])
    w_ref[...] = (1.0 - z) * n + z * x0


_gru = pl.pallas_call(
    _gru_body,
    out_shape=jax.ShapeDtypeStruct((D, D), jnp.float32),
)


_YBLK = 1000


def _y_body(x_ref, w_ref, degt_ref, yh_ref):
    dn = (((1,), (0,)), ((), ()))
    xw = lax.dot_general(x_ref[...], w_ref[...], dn,
                         preferred_element_type=jnp.float32)
    dp = degt_ref[...]
    dinv = lax.rsqrt(dp[:, 0:1] + dp[:, 1:2] + 1.0)
    y = dinv * xw
    yh_ref[0] = y[:, 0:HD]
    yh_ref[1] = y[:, HD:D]


_y_call = pl.pallas_call(
    _y_body,
    grid=(N // _YBLK,),
    in_specs=[
        pl.BlockSpec((_YBLK, D), lambda i: (i, 0)),
        pl.BlockSpec((D, D), lambda i: (0, 0)),
        pl.BlockSpec((_YBLK, 2), lambda i: (i, 0)),
    ],
    out_specs=pl.BlockSpec((2, _YBLK, HD), lambda i: (0, i, 0)),
    out_shape=jax.ShapeDtypeStruct((NC, N, HD), jnp.float32),
)


_CBLK = 1000


def _comb_body(acc_ref, degt_ref, out_ref):
    dp = degt_ref[...]
    dinv = lax.rsqrt(dp[:, 0:1] + dp[:, 1:2] + 1.0)
    out_ref[...] = dinv * jnp.concatenate([acc_ref[0], acc_ref[1]], axis=1)


_comb_call = pl.pallas_call(
    _comb_body,
    grid=(N // _CBLK,),
    in_specs=[
        pl.BlockSpec((2, _CBLK, HD), lambda i: (0, i, 0)),
        pl.BlockSpec((_CBLK, 2), lambda i: (i, 0)),
    ],
    out_specs=pl.BlockSpec((_CBLK, D), lambda i: (i, 0)),
    out_shape=jax.ShapeDtypeStruct((N, D), jnp.float32),
)


def kernel(edge_index, X, initial_weight, W_ih, W_hh, b_ih, b_hh):
    row32 = edge_index[0].astype(jnp.int32)
    col32 = edge_index[1].astype(jnp.int32)
    row_w = row32.reshape(NW, CH1, CHW)
    col_w = col32.reshape(NW, CH1, CHW)
    pad = EPTP - E // NS
    row_t = jnp.concatenate(
        [row32.reshape(NS, E // NS),
         jnp.zeros((NS, pad), jnp.int32)], axis=1).reshape(NS, NCH, CHS)
    col_t = jnp.concatenate(
        [col32.reshape(NS, E // NS),
         jnp.full((NS, pad), TRASH, jnp.int32)], axis=1).reshape(
             NS, NCH, CHS)

    W = _gru(initial_weight[0], W_ih, W_hh,
             b_ih.reshape(1, 3 * D), b_hh.reshape(1, 3 * D))

    deg_parts = _deg_kernel(col_w)
    degt = deg_parts.reshape(NC, DEG_N)[:, :N].T

    yh = _y_call(X, W, degt)
    acc = _prop_kernel(row_t, col_t, yh)
    out = _comb_call(acc, degt)
    return out


# speedup vs baseline: 2.4700x; 1.1644x over previous
import functools

import jax
import jax.numpy as jnp
from jax import lax
from jax.experimental import pallas as pl
from jax.experimental.pallas import tpu as pltpu
from jax.experimental.pallas import tpu_sc as plsc

N = 10000
E = 320000
D = 128
HD = D // 2

NC = 2
NS = 16
NW = NC * NS
CHW = 80

EPW = E // NW
CH1 = EPW // CHW

TRASH = N
ACC_ROWS = N + 8
RPT = 632
RPT_LAST = N - RPT * (NS - 1)

DEG_RPT = 640
DEG_N = NS * DEG_RPT

_mesh = plsc.VectorSubcoreMesh(core_axis_name="c", subcore_axis_name="s")


@functools.partial(
    pl.kernel,
    out_type=jax.ShapeDtypeStruct((NC * DEG_N,), jnp.float32),
    mesh=_mesh,
    scratch_types=[
        pltpu.VMEM((CH1, CHW), jnp.int32),
        pltpu.VMEM((CHW,), jnp.float32),
        pltpu.VMEM((DEG_RPT,), jnp.float32),
        pltpu.VMEM_SHARED((DEG_N,), jnp.float32),
    ],
)
def _deg_kernel(col_hbm, deg_out, col_v, ones_v, zero_v, deg_acc):
    c = lax.axis_index("c")
    s = lax.axis_index("s")
    w = s * NC + c

    pltpu.sync_copy(col_hbm.at[w], col_v)
    for i in range(CHW // 16):
        ones_v[pl.ds(i * 16, 16)] = jnp.ones((16,), jnp.float32)
    for i in range(DEG_RPT // 16):
        zero_v[pl.ds(i * 16, 16)] = jnp.zeros((16,), jnp.float32)
    pltpu.sync_copy(zero_v, deg_acc.at[pl.ds(s * DEG_RPT, DEG_RPT)])
    plsc.subcore_barrier()

    def body(j, _):
        pltpu.sync_copy(ones_v, deg_acc.at[col_v.at[j]], add=True)
        return 0

    lax.fori_loop(0, CH1, body, 0)
    plsc.subcore_barrier()
    pltpu.sync_copy(deg_acc.at[pl.ds(s * DEG_RPT, DEG_RPT)],
                    deg_out.at[pl.ds(c * DEG_N + s * DEG_RPT, DEG_RPT)])


CHS = 128
NCH = 160
NCH2 = NCH // 2
EPTP = NCH * CHS


@functools.partial(
    pl.kernel,
    out_type=jax.ShapeDtypeStruct((NC, N, HD), jnp.float32),
    mesh=_mesh,
    compiler_params=pltpu.CompilerParams(needs_layout_passes=False,
                                         use_tc_tiling_on_sc=False),
    scratch_types=[
        pltpu.VMEM((NCH2, CHS), jnp.int32),
        pltpu.VMEM((NCH2, CHS), jnp.int32),
        pltpu.VMEM((CHS, HD), jnp.float32),
        pltpu.VMEM_SHARED((N, HD), jnp.float32),
        pltpu.VMEM_SHARED((ACC_ROWS, HD), jnp.float32),
        pltpu.SemaphoreType.DMA,
    ],
)
def _prop_kernel(row_hbm, col_hbm, yh_hbm, acc_out,
                 row_v, col_v, buf0, ysp, acc, sem0):
    c = lax.axis_index("c")
    s = lax.axis_index("s")
    base = s * RPT

    @pl.when(s < NS - 1)
    def _():
        pltpu.sync_copy(yh_hbm.at[c].at[pl.ds(base, RPT)],
                        ysp.at[pl.ds(base, RPT)])
        pltpu.sync_copy(yh_hbm.at[c].at[pl.ds(base, RPT)],
                        acc.at[pl.ds(base, RPT)])

    @pl.when(s == NS - 1)
    def _():
        pltpu.sync_copy(yh_hbm.at[c].at[pl.ds(base, RPT_LAST)],
                        ysp.at[pl.ds(base, RPT_LAST)])
        pltpu.sync_copy(yh_hbm.at[c].at[pl.ds(base, RPT_LAST)],
                        acc.at[pl.ds(base, RPT_LAST)])

    plsc.subcore_barrier()

    for h in range(2):
        pltpu.sync_copy(row_hbm.at[s].at[pl.ds(h * NCH2, NCH2)], row_v)
        pltpu.sync_copy(col_hbm.at[s].at[pl.ds(h * NCH2, NCH2)], col_v)

        def body(j, _):
            pltpu.async_copy(ysp.at[row_v.at[j]], buf0, sem0).wait()
            pltpu.sync_copy(buf0, acc.at[col_v.at[j]], add=True)
            return 0

        lax.fori_loop(0, NCH2, body, 0)

    plsc.subcore_barrier()

    @pl.when(s < NS - 1)
    def _():
        pltpu.sync_copy(acc.at[pl.ds(base, RPT)],
                        acc_out.at[c].at[pl.ds(base, RPT)])

    @pl.when(s == NS - 1)
    def _():
        pltpu.sync_copy(acc.at[pl.ds(base, RPT_LAST)],
                        acc_out.at[c].at[pl.ds(base, RPT_LAST)])


def _gru_body(x0_ref, wih_ref, whh_ref, bih_ref, bhh_ref, w_ref):
    x0 = x0_ref[...]
    dn = (((1,), (1,)), ((), ()))
    gi = lax.dot_general(x0, wih_ref[...], dn,
                         preferred_element_type=jnp.float32) + bih_ref[...]
    gh = lax.dot_general(x0, whh_ref[...], dn,
                         preferred_element_type=jnp.float32) + bhh_ref[...]
    r = jax.nn.sigmoid(gi[:, 0:D] + gh[:, 0:D])
    z = jax.nn.sigmoid(gi[:, D:2 * D] + gh[:, D:2 * D])
    n = jnp.tanh(gi[:, 2 * D:3 * D] + r * gh[:, 2 * D:3 * D])
    w_ref[...] = (1.0 - z) * n + z * x0


_gru = pl.pallas_call(
    _gru_body,
    out_shape=jax.ShapeDtypeStruct((D, D), jnp.float32),
)


_YBLK = 1000


def _y_body(x_ref, w_ref, degt_ref, yh_ref):
    dn = (((1,), (0,)), ((), ()))
    xw = lax.dot_general(x_ref[...], w_ref[...], dn,
                         preferred_element_type=jnp.float32)
    dp = degt_ref[...]
    dinv = lax.rsqrt(dp[:, 0:1] + dp[:, 1:2] + 1.0)
    y = dinv * xw
    yh_ref[0] = y[:, 0:HD]
    yh_ref[1] = y[:, HD:D]


_y_call = pl.pallas_call(
    _y_body,
    grid=(N // _YBLK,),
    in_specs=[
        pl.BlockSpec((_YBLK, D), lambda i: (i, 0)),
        pl.BlockSpec((D, D), lambda i: (0, 0)),
        pl.BlockSpec((_YBLK, 2), lambda i: (i, 0)),
    ],
    out_specs=pl.BlockSpec((2, _YBLK, HD), lambda i: (0, i, 0)),
    out_shape=jax.ShapeDtypeStruct((NC, N, HD), jnp.float32),
)


_CBLK = 1000


def _comb_body(acc_ref, degt_ref, out_ref):
    dp = degt_ref[...]
    dinv = lax.rsqrt(dp[:, 0:1] + dp[:, 1:2] + 1.0)
    out_ref[...] = dinv * jnp.concatenate([acc_ref[0], acc_ref[1]], axis=1)


_comb_call = pl.pallas_call(
    _comb_body,
    grid=(N // _CBLK,),
    in_specs=[
        pl.BlockSpec((2, _CBLK, HD), lambda i: (0, i, 0)),
        pl.BlockSpec((_CBLK, 2), lambda i: (i, 0)),
    ],
    out_specs=pl.BlockSpec((_CBLK, D), lambda i: (i, 0)),
    out_shape=jax.ShapeDtypeStruct((N, D), jnp.float32),
)


def kernel(edge_index, X, initial_weight, W_ih, W_hh, b_ih, b_hh):
    row32 = edge_index[0].astype(jnp.int32)
    col32 = edge_index[1].astype(jnp.int32)
    row_w = row32.reshape(NW, CH1, CHW)
    col_w = col32.reshape(NW, CH1, CHW)
    pad = EPTP - E // NS
    row_t = jnp.concatenate(
        [row32.reshape(NS, E // NS),
         jnp.zeros((NS, pad), jnp.int32)], axis=1).reshape(NS, NCH, CHS)
    col_t = jnp.concatenate(
        [col32.reshape(NS, E // NS),
         jnp.full((NS, pad), TRASH, jnp.int32)], axis=1).reshape(
             NS, NCH, CHS)

    W = _gru(initial_weight[0], W_ih, W_hh,
             b_ih.reshape(1, 3 * D), b_hh.reshape(1, 3 * D))

    deg_parts = _deg_kernel(col_w)
    degt = deg_parts.reshape(NC, DEG_N)[:, :N].T

    yh = _y_call(X, W, degt)
    acc = _prop_kernel(row_t, col_t, yh)
    out = _comb_call(acc, degt)
    return out


# speedup vs baseline: 2.4884x; 1.0074x over previous
import functools

import jax
import jax.numpy as jnp
from jax import lax
from jax.experimental import pallas as pl
from jax.experimental.pallas import tpu as pltpu
from jax.experimental.pallas import tpu_sc as plsc

N = 10000
E = 320000
D = 128
HD = D // 2

NC = 2
NS = 16
NW = NC * NS
CHW = 80

EPW = E // NW
CH1 = EPW // CHW

TRASH = N
ACC_ROWS = N + 8
RPT = 632
RPT_LAST = N - RPT * (NS - 1)

DEG_RPT = 640
DEG_N = NS * DEG_RPT

_mesh = plsc.VectorSubcoreMesh(core_axis_name="c", subcore_axis_name="s")


@functools.partial(
    pl.kernel,
    out_type=jax.ShapeDtypeStruct((NC * DEG_N,), jnp.float32),
    mesh=_mesh,
    scratch_types=[
        pltpu.VMEM((CH1, CHW), jnp.int32),
        pltpu.VMEM((CHW,), jnp.float32),
        pltpu.VMEM((DEG_RPT,), jnp.float32),
        pltpu.VMEM_SHARED((DEG_N,), jnp.float32),
    ],
)
def _deg_kernel(col_hbm, deg_out, col_v, ones_v, zero_v, deg_acc):
    c = lax.axis_index("c")
    s = lax.axis_index("s")
    w = s * NC + c

    pltpu.sync_copy(col_hbm.at[w], col_v)
    for i in range(CHW // 16):
        ones_v[pl.ds(i * 16, 16)] = jnp.ones((16,), jnp.float32)
    for i in range(DEG_RPT // 16):
        zero_v[pl.ds(i * 16, 16)] = jnp.zeros((16,), jnp.float32)
    pltpu.sync_copy(zero_v, deg_acc.at[pl.ds(s * DEG_RPT, DEG_RPT)])
    plsc.subcore_barrier()

    def body(j, _):
        pltpu.sync_copy(ones_v, deg_acc.at[col_v.at[j]], add=True)
        return 0

    lax.fori_loop(0, CH1, body, 0)
    plsc.subcore_barrier()
    pltpu.sync_copy(deg_acc.at[pl.ds(s * DEG_RPT, DEG_RPT)],
                    deg_out.at[pl.ds(c * DEG_N + s * DEG_RPT, DEG_RPT)])


CHS = 256
NCH = 80
NCH2 = NCH // 2
EPTP = NCH * CHS


@functools.partial(
    pl.kernel,
    out_type=jax.ShapeDtypeStruct((NC, N, HD), jnp.float32),
    mesh=_mesh,
    compiler_params=pltpu.CompilerParams(needs_layout_passes=False,
                                         use_tc_tiling_on_sc=False),
    scratch_types=[
        pltpu.VMEM((NCH2, CHS), jnp.int32),
        pltpu.VMEM((NCH2, CHS), jnp.int32),
        pltpu.VMEM((CHS, HD), jnp.float32),
        pltpu.VMEM_SHARED((N, HD), jnp.float32),
        pltpu.VMEM_SHARED((ACC_ROWS, HD), jnp.float32),
        pltpu.SemaphoreType.DMA,
    ],
)
def _prop_kernel(row_hbm, col_hbm, yh_hbm, acc_out,
                 row_v, col_v, buf0, ysp, acc, sem0):
    c = lax.axis_index("c")
    s = lax.axis_index("s")
    base = s * RPT

    @pl.when(s < NS - 1)
    def _():
        pltpu.sync_copy(yh_hbm.at[c].at[pl.ds(base, RPT)],
                        ysp.at[pl.ds(base, RPT)])
        pltpu.sync_copy(yh_hbm.at[c].at[pl.ds(base, RPT)],
                        acc.at[pl.ds(base, RPT)])

    @pl.when(s == NS - 1)
    def _():
        pltpu.sync_copy(yh_hbm.at[c].at[pl.ds(base, RPT_LAST)],
                        ysp.at[pl.ds(base, RPT_LAST)])
        pltpu.sync_copy(yh_hbm.at[c].at[pl.ds(base, RPT_LAST)],
                        acc.at[pl.ds(base, RPT_LAST)])

    plsc.subcore_barrier()

    for h in range(2):
        pltpu.sync_copy(row_hbm.at[s].at[pl.ds(h * NCH2, NCH2)], row_v)
        pltpu.sync_copy(col_hbm.at[s].at[pl.ds(h * NCH2, NCH2)], col_v)

        def body(j, _):
            pltpu.async_copy(ysp.at[row_v.at[j]], buf0, sem0).wait()
            pltpu.sync_copy(buf0, acc.at[col_v.at[j]], add=True)
            return 0

        lax.fori_loop(0, NCH2, body, 0)

    plsc.subcore_barrier()

    @pl.when(s < NS - 1)
    def _():
        pltpu.sync_copy(acc.at[pl.ds(base, RPT)],
                        acc_out.at[c].at[pl.ds(base, RPT)])

    @pl.when(s == NS - 1)
    def _():
        pltpu.sync_copy(acc.at[pl.ds(base, RPT_LAST)],
                        acc_out.at[c].at[pl.ds(base, RPT_LAST)])


def _gru_body(x0_ref, wih_ref, whh_ref, bih_ref, bhh_ref, w_ref):
    x0 = x0_ref[...]
    dn = (((1,), (1,)), ((), ()))
    gi = lax.dot_general(x0, wih_ref[...], dn,
                         preferred_element_type=jnp.float32) + bih_ref[...]
    gh = lax.dot_general(x0, whh_ref[...], dn,
                         preferred_element_type=jnp.float32) + bhh_ref[...]
    r = jax.nn.sigmoid(gi[:, 0:D] + gh[:, 0:D])
    z = jax.nn.sigmoid(gi[:, D:2 * D] + gh[:, D:2 * D])
    n = jnp.tanh(gi[:, 2 * D:3 * D] + r * gh[:, 2 * D:3 * D])
    w_ref[...] = (1.0 - z) * n + z * x0


_gru = pl.pallas_call(
    _gru_body,
    out_shape=jax.ShapeDtypeStruct((D, D), jnp.float32),
)


_YBLK = 1000


def _y_body(x_ref, w_ref, degt_ref, yh_ref):
    dn = (((1,), (0,)), ((), ()))
    xw = lax.dot_general(x_ref[...], w_ref[...], dn,
                         preferred_element_type=jnp.float32)
    dp = degt_ref[...]
    dinv = lax.rsqrt(dp[:, 0:1] + dp[:, 1:2] + 1.0)
    y = dinv * xw
    yh_ref[0] = y[:, 0:HD]
    yh_ref[1] = y[:, HD:D]


_y_call = pl.pallas_call(
    _y_body,
    grid=(N // _YBLK,),
    in_specs=[
        pl.BlockSpec((_YBLK, D), lambda i: (i, 0)),
        pl.BlockSpec((D, D), lambda i: (0, 0)),
        pl.BlockSpec((_YBLK, 2), lambda i: (i, 0)),
    ],
    out_specs=pl.BlockSpec((2, _YBLK, HD), lambda i: (0, i, 0)),
    out_shape=jax.ShapeDtypeStruct((NC, N, HD), jnp.float32),
)


_CBLK = 1000


def _comb_body(acc_ref, degt_ref, out_ref):
    dp = degt_ref[...]
    dinv = lax.rsqrt(dp[:, 0:1] + dp[:, 1:2] + 1.0)
    out_ref[...] = dinv * jnp.concatenate([acc_ref[0], acc_ref[1]], axis=1)


_comb_call = pl.pallas_call(
    _comb_body,
    grid=(N // _CBLK,),
    in_specs=[
        pl.BlockSpec((2, _CBLK, HD), lambda i: (0, i, 0)),
        pl.BlockSpec((_CBLK, 2), lambda i: (i, 0)),
    ],
    out_specs=pl.BlockSpec((_CBLK, D), lambda i: (i, 0)),
    out_shape=jax.ShapeDtypeStruct((N, D), jnp.float32),
)


def kernel(edge_index, X, initial_weight, W_ih, W_hh, b_ih, b_hh):
    row32 = edge_index[0].astype(jnp.int32)
    col32 = edge_index[1].astype(jnp.int32)
    row_w = row32.reshape(NW, CH1, CHW)
    col_w = col32.reshape(NW, CH1, CHW)
    pad = EPTP - E // NS
    row_t = jnp.concatenate(
        [row32.reshape(NS, E // NS),
         jnp.zeros((NS, pad), jnp.int32)], axis=1).reshape(NS, NCH, CHS)
    col_t = jnp.concatenate(
        [col32.reshape(NS, E // NS),
         jnp.full((NS, pad), TRASH, jnp.int32)], axis=1).reshape(
             NS, NCH, CHS)

    W = _gru(initial_weight[0], W_ih, W_hh,
             b_ih.reshape(1, 3 * D), b_hh.reshape(1, 3 * D))

    deg_parts = _deg_kernel(col_w)
    degt = deg_parts.reshape(NC, DEG_N)[:, :N].T

    yh = _y_call(X, W, degt)
    acc = _prop_kernel(row_t, col_t, yh)
    out = _comb_call(acc, degt)
    return out


# speedup vs baseline: 3.1123x; 1.2507x over previous
import functools

import jax
import jax.numpy as jnp
from jax import lax
from jax.experimental import pallas as pl
from jax.experimental.pallas import tpu as pltpu
from jax.experimental.pallas import tpu_sc as plsc

N = 10000
E = 320000
D = 128
HD = D // 2

NC = 2
NS = 16
NW = NC * NS
CHW = 80

EPW = E // NW
CH1 = EPW // CHW

TRASH = N
ACC_ROWS = N + 8
RPT = 632
RPT_LAST = N - RPT * (NS - 1)

DEG_RPT = 640
DEG_N = NS * DEG_RPT

_mesh = plsc.VectorSubcoreMesh(core_axis_name="c", subcore_axis_name="s")


@functools.partial(
    pl.kernel,
    out_type=jax.ShapeDtypeStruct((NC * DEG_N,), jnp.float32),
    mesh=_mesh,
    scratch_types=[
        pltpu.VMEM((CH1, CHW), jnp.int32),
        pltpu.VMEM((CHW,), jnp.float32),
        pltpu.VMEM((DEG_RPT,), jnp.float32),
        pltpu.VMEM_SHARED((DEG_N,), jnp.float32),
    ],
)
def _deg_kernel(col_hbm, deg_out, col_v, ones_v, zero_v, deg_acc):
    c = lax.axis_index("c")
    s = lax.axis_index("s")
    w = s * NC + c

    pltpu.sync_copy(col_hbm.at[w], col_v)
    for i in range(CHW // 16):
        ones_v[pl.ds(i * 16, 16)] = jnp.ones((16,), jnp.float32)
    for i in range(DEG_RPT // 16):
        zero_v[pl.ds(i * 16, 16)] = jnp.zeros((16,), jnp.float32)
    pltpu.sync_copy(zero_v, deg_acc.at[pl.ds(s * DEG_RPT, DEG_RPT)])
    plsc.subcore_barrier()

    def body(j, _):
        pltpu.sync_copy(ones_v, deg_acc.at[col_v.at[j]], add=True)
        return 0

    lax.fori_loop(0, CH1, body, 0)
    plsc.subcore_barrier()
    pltpu.sync_copy(deg_acc.at[pl.ds(s * DEG_RPT, DEG_RPT)],
                    deg_out.at[pl.ds(c * DEG_N + s * DEG_RPT, DEG_RPT)])


CHS = 128
NCH = 160
NCH2 = NCH // 2
EPTP = NCH * CHS


@functools.partial(
    pl.kernel,
    out_type=jax.ShapeDtypeStruct((NC, N, HD), jnp.float32),
    mesh=_mesh,
    compiler_params=pltpu.CompilerParams(needs_layout_passes=False,
                                         use_tc_tiling_on_sc=False),
    scratch_types=[
        pltpu.VMEM((NCH2, CHS), jnp.int32),
        pltpu.VMEM((NCH2, CHS), jnp.int32),
        pltpu.VMEM((CHS, HD), jnp.float32),
        pltpu.VMEM((CHS, HD), jnp.float32),
        pltpu.VMEM_SHARED((N, HD), jnp.float32),
        pltpu.VMEM_SHARED((ACC_ROWS, HD), jnp.float32),
        pltpu.SemaphoreType.DMA,
        pltpu.SemaphoreType.DMA,
    ],
)
def _prop_kernel(row_hbm, col_hbm, yh_hbm, acc_out,
                 row_v, col_v, buf0, buf1, ysp, acc, sem0, sem1):
    c = lax.axis_index("c")
    s = lax.axis_index("s")
    base = s * RPT

    @pl.when(s < NS - 1)
    def _():
        pltpu.sync_copy(yh_hbm.at[c].at[pl.ds(base, RPT)],
                        ysp.at[pl.ds(base, RPT)])
        pltpu.sync_copy(yh_hbm.at[c].at[pl.ds(base, RPT)],
                        acc.at[pl.ds(base, RPT)])

    @pl.when(s == NS - 1)
    def _():
        pltpu.sync_copy(yh_hbm.at[c].at[pl.ds(base, RPT_LAST)],
                        ysp.at[pl.ds(base, RPT_LAST)])
        pltpu.sync_copy(yh_hbm.at[c].at[pl.ds(base, RPT_LAST)],
                        acc.at[pl.ds(base, RPT_LAST)])

    plsc.subcore_barrier()

    def gather(j, buf, sem):
        return pltpu.make_async_copy(ysp.at[row_v.at[j]], buf, sem)

    for h in range(2):
        pltpu.sync_copy(row_hbm.at[s].at[pl.ds(h * NCH2, NCH2)], row_v)
        pltpu.sync_copy(col_hbm.at[s].at[pl.ds(h * NCH2, NCH2)], col_v)

        gather(0, buf0, sem0).start()

        def body(i, _):
            j = 2 * i
            gather(j, buf0, sem0).wait()
            gather(j + 1, buf1, sem1).start()
            pltpu.sync_copy(buf0, acc.at[col_v.at[j]], add=True)
            gather(j + 1, buf1, sem1).wait()
            gather(j + 2, buf0, sem0).start()
            pltpu.sync_copy(buf1, acc.at[col_v.at[j + 1]], add=True)
            return 0

        lax.fori_loop(0, NCH2 // 2 - 1, body, 0)
        gather(NCH2 - 2, buf0, sem0).wait()
        gather(NCH2 - 1, buf1, sem1).start()
        pltpu.sync_copy(buf0, acc.at[col_v.at[NCH2 - 2]], add=True)
        gather(NCH2 - 1, buf1, sem1).wait()
        pltpu.sync_copy(buf1, acc.at[col_v.at[NCH2 - 1]], add=True)

    plsc.subcore_barrier()

    @pl.when(s < NS - 1)
    def _():
        pltpu.sync_copy(acc.at[pl.ds(base, RPT)],
                        acc_out.at[c].at[pl.ds(base, RPT)])

    @pl.when(s == NS - 1)
    def _():
        pltpu.sync_copy(acc.at[pl.ds(base, RPT_LAST)],
                        acc_out.at[c].at[pl.ds(base, RPT_LAST)])


def _gru_body(x0_ref, wih_ref, whh_ref, bih_ref, bhh_ref, w_ref):
    x0 = x0_ref[...]
    dn = (((1,), (1,)), ((), ()))
    gi = lax.dot_general(x0, wih_ref[...], dn,
                         preferred_element_type=jnp.float32) + bih_ref[...]
    gh = lax.dot_general(x0, whh_ref[...], dn,
                         preferred_element_type=jnp.float32) + bhh_ref[...]
    r = jax.nn.sigmoid(gi[:, 0:D] + gh[:, 0:D])
    z = jax.nn.sigmoid(gi[:, D:2 * D] + gh[:, D:2 * D])
    n = jnp.tanh(gi[:, 2 * D:3 * D] + r * gh[:, 2 * D:3 * D])
    w_ref[...] = (1.0 - z) * n + z * x0


_gru = pl.pallas_call(
    _gru_body,
    out_shape=jax.ShapeDtypeStruct((D, D), jnp.float32),
)


_YBLK = 1000


def _y_body(x_ref, w_ref, degt_ref, yh_ref):
    dn = (((1,), (0,)), ((), ()))
    xw = lax.dot_general(x_ref[...], w_ref[...], dn,
                         preferred_element_type=jnp.float32)
    dp = degt_ref[...]
    dinv = lax.rsqrt(dp[:, 0:1] + dp[:, 1:2] + 1.0)
    y = dinv * xw
    yh_ref[0] = y[:, 0:HD]
    yh_ref[1] = y[:, HD:D]


_y_call = pl.pallas_call(
    _y_body,
    grid=(N // _YBLK,),
    in_specs=[
        pl.BlockSpec((_YBLK, D), lambda i: (i, 0)),
        pl.BlockSpec((D, D), lambda i: (0, 0)),
        pl.BlockSpec((_YBLK, 2), lambda i: (i, 0)),
    ],
    out_specs=pl.BlockSpec((2, _YBLK, HD), lambda i: (0, i, 0)),
    out_shape=jax.ShapeDtypeStruct((NC, N, HD), jnp.float32),
)


_CBLK = 1000


def _comb_body(acc_ref, degt_ref, out_ref):
    dp = degt_ref[...]
    dinv = lax.rsqrt(dp[:, 0:1] + dp[:, 1:2] + 1.0)
    out_ref[...] = dinv * jnp.concatenate([acc_ref[0], acc_ref[1]], axis=1)


_comb_call = pl.pallas_call(
    _comb_body,
    grid=(N // _CBLK,),
    in_specs=[
        pl.BlockSpec((2, _CBLK, HD), lambda i: (0, i, 0)),
        pl.BlockSpec((_CBLK, 2), lambda i: (i, 0)),
    ],
    out_specs=pl.BlockSpec((_CBLK, D), lambda i: (i, 0)),
    out_shape=jax.ShapeDtypeStruct((N, D), jnp.float32),
)


def kernel(edge_index, X, initial_weight, W_ih, W_hh, b_ih, b_hh):
    row32 = edge_index[0].astype(jnp.int32)
    col32 = edge_index[1].astype(jnp.int32)
    row_w = row32.reshape(NW, CH1, CHW)
    col_w = col32.reshape(NW, CH1, CHW)
    pad = EPTP - E // NS
    row_t = jnp.concatenate(
        [row32.reshape(NS, E // NS),
         jnp.zeros((NS, pad), jnp.int32)], axis=1).reshape(NS, NCH, CHS)
    col_t = jnp.concatenate(
        [col32.reshape(NS, E // NS),
         jnp.full((NS, pad), TRASH, jnp.int32)], axis=1).reshape(
             NS, NCH, CHS)

    W = _gru(initial_weight[0], W_ih, W_hh,
             b_ih.reshape(1, 3 * D), b_hh.reshape(1, 3 * D))

    deg_parts = _deg_kernel(col_w)
    degt = deg_parts.reshape(NC, DEG_N)[:, :N].T

    yh = _y_call(X, W, degt)
    acc = _prop_kernel(row_t, col_t, yh)
    out = _comb_call(acc, degt)
    return out


# speedup vs baseline: 3.4070x; 1.0947x over previous
import functools

import jax
import jax.numpy as jnp
from jax import lax
from jax.experimental import pallas as pl
from jax.experimental.pallas import tpu as pltpu
from jax.experimental.pallas import tpu_sc as plsc

N = 10000
E = 320000
D = 128
HD = D // 2

NC = 2
NS = 16
NW = NC * NS
CHW = 80

EPW = E // NW
CH1 = EPW // CHW

TRASH = N
ACC_ROWS = N + 8
RPT = 632
RPT_LAST = N - RPT * (NS - 1)

DEG_RPT = 640
DEG_N = NS * DEG_RPT

_mesh = plsc.VectorSubcoreMesh(core_axis_name="c", subcore_axis_name="s")


@functools.partial(
    pl.kernel,
    out_type=jax.ShapeDtypeStruct((NC * DEG_N,), jnp.float32),
    mesh=_mesh,
    scratch_types=[
        pltpu.VMEM((CH1, CHW), jnp.int32),
        pltpu.VMEM((CHW,), jnp.float32),
        pltpu.VMEM((DEG_RPT,), jnp.float32),
        pltpu.VMEM_SHARED((DEG_N,), jnp.float32),
    ],
)
def _deg_kernel(col_hbm, deg_out, col_v, ones_v, zero_v, deg_acc):
    c = lax.axis_index("c")
    s = lax.axis_index("s")
    w = s * NC + c

    pltpu.sync_copy(col_hbm.at[w], col_v)
    for i in range(CHW // 16):
        ones_v[pl.ds(i * 16, 16)] = jnp.ones((16,), jnp.float32)
    for i in range(DEG_RPT // 16):
        zero_v[pl.ds(i * 16, 16)] = jnp.zeros((16,), jnp.float32)
    pltpu.sync_copy(zero_v, deg_acc.at[pl.ds(s * DEG_RPT, DEG_RPT)])
    plsc.subcore_barrier()

    def body(j, _):
        pltpu.sync_copy(ones_v, deg_acc.at[col_v.at[j]], add=True)
        return 0

    lax.fori_loop(0, CH1, body, 0)
    plsc.subcore_barrier()
    pltpu.sync_copy(deg_acc.at[pl.ds(s * DEG_RPT, DEG_RPT)],
                    deg_out.at[pl.ds(c * DEG_N + s * DEG_RPT, DEG_RPT)])


CHS = 128
NCH = 160
NST = 4
NCH2 = NCH // NST
EPTP = NCH * CHS


@functools.partial(
    pl.kernel,
    out_type=jax.ShapeDtypeStruct((NC, N, HD), jnp.float32),
    mesh=_mesh,
    compiler_params=pltpu.CompilerParams(needs_layout_passes=False,
                                         use_tc_tiling_on_sc=False),
    scratch_types=[
        pltpu.VMEM((NCH2, CHS), jnp.int32),
        pltpu.VMEM((NCH2, CHS), jnp.int32),
        pltpu.VMEM((CHS, HD), jnp.float32),
        pltpu.VMEM((CHS, HD), jnp.float32),
        pltpu.VMEM((CHS, HD), jnp.float32),
        pltpu.VMEM((CHS, HD), jnp.float32),
        pltpu.VMEM_SHARED((N, HD), jnp.float32),
        pltpu.VMEM_SHARED((ACC_ROWS, HD), jnp.float32),
        pltpu.SemaphoreType.DMA,
        pltpu.SemaphoreType.DMA,
        pltpu.SemaphoreType.DMA,
        pltpu.SemaphoreType.DMA,
        pltpu.SemaphoreType.DMA,
        pltpu.SemaphoreType.DMA,
        pltpu.SemaphoreType.DMA,
        pltpu.SemaphoreType.DMA,
    ],
)
def _prop_kernel(row_hbm, col_hbm, yh_hbm, acc_out,
                 row_v, col_v, buf0, buf1, buf2, buf3, ysp, acc,
                 gs0, gs1, gs2, gs3, ss0, ss1, ss2, ss3):
    c = lax.axis_index("c")
    s = lax.axis_index("s")
    base = s * RPT

    @pl.when(s < NS - 1)
    def _():
        pltpu.sync_copy(yh_hbm.at[c].at[pl.ds(base, RPT)],
                        ysp.at[pl.ds(base, RPT)])
        pltpu.sync_copy(yh_hbm.at[c].at[pl.ds(base, RPT)],
                        acc.at[pl.ds(base, RPT)])

    @pl.when(s == NS - 1)
    def _():
        pltpu.sync_copy(yh_hbm.at[c].at[pl.ds(base, RPT_LAST)],
                        ysp.at[pl.ds(base, RPT_LAST)])
        pltpu.sync_copy(yh_hbm.at[c].at[pl.ds(base, RPT_LAST)],
                        acc.at[pl.ds(base, RPT_LAST)])

    plsc.subcore_barrier()

    bufs = [buf0, buf1, buf2, buf3]
    gsem = [gs0, gs1, gs2, gs3]
    ssem = [ss0, ss1, ss2, ss3]

    def gather(j, p):
        return pltpu.make_async_copy(ysp.at[row_v.at[j]], bufs[p], gsem[p])

    def scatter(j, p):
        return pltpu.make_async_copy(bufs[p], acc.at[col_v.at[j]], ssem[p])

    def step(j, p, wait_scatter):
        gather(j, p).wait()
        if wait_scatter:
            scatter(j - 2, (p + 2) % 4).wait()
        gather(j + 2, (p + 2) % 4).start()
        pltpu.async_copy(bufs[p], acc.at[col_v.at[j]], ssem[p], add=True)

    for h in range(NST):
        pltpu.sync_copy(row_hbm.at[s].at[pl.ds(h * NCH2, NCH2)], row_v)
        pltpu.sync_copy(col_hbm.at[s].at[pl.ds(h * NCH2, NCH2)], col_v)

        gather(0, 0).start()
        gather(1, 1).start()
        step(0, 0, False)
        step(1, 1, False)
        step(2, 2, True)
        step(3, 3, True)

        def body(i, _):
            j = 4 * i
            for p in range(4):
                step(j + p, p, True)
            return 0

        lax.fori_loop(1, NCH2 // 4 - 1, body, 0)
        jl = NCH2 - 4
        step(jl, 0, True)
        step(jl + 1, 1, True)
        gather(jl + 2, 2).wait()
        scatter(jl, 0).wait()
        pltpu.async_copy(bufs[2], acc.at[col_v.at[jl + 2]], ssem[2], add=True)
        gather(jl + 3, 3).wait()
        scatter(jl + 1, 1).wait()
        pltpu.async_copy(bufs[3], acc.at[col_v.at[jl + 3]], ssem[3], add=True)
        scatter(jl + 2, 2).wait()
        scatter(jl + 3, 3).wait()

    plsc.subcore_barrier()

    @pl.when(s < NS - 1)
    def _():
        pltpu.sync_copy(acc.at[pl.ds(base, RPT)],
                        acc_out.at[c].at[pl.ds(base, RPT)])

    @pl.when(s == NS - 1)
    def _():
        pltpu.sync_copy(acc.at[pl.ds(base, RPT_LAST)],
                        acc_out.at[c].at[pl.ds(base, RPT_LAST)])


def _gru_body(x0_ref, wih_ref, whh_ref, bih_ref, bhh_ref, w_ref):
    x0 = x0_ref[...]
    dn = (((1,), (1,)), ((), ()))
    gi = lax.dot_general(x0, wih_ref[...], dn,
                         preferred_element_type=jnp.float32) + bih_ref[...]
    gh = lax.dot_general(x0, whh_ref[...], dn,
                         preferred_element_type=jnp.float32) + bhh_ref[...]
    r = jax.nn.sigmoid(gi[:, 0:D] + gh[:, 0:D])
    z = jax.nn.sigmoid(gi[:, D:2 * D] + gh[:, D:2 * D])
    n = jnp.tanh(gi[:, 2 * D:3 * D] + r * gh[:, 2 * D:3 * D])
    w_ref[...] = (1.0 - z) * n + z * x0


_gru = pl.pallas_call(
    _gru_body,
    out_shape=jax.ShapeDtypeStruct((D, D), jnp.float32),
)


_YBLK = 1000


def _y_body(x_ref, w_ref, degt_ref, yh_ref):
    dn = (((1,), (0,)), ((), ()))
    xw = lax.dot_general(x_ref[...], w_ref[...], dn,
                         preferred_element_type=jnp.float32)
    dp = degt_ref[...]
    dinv = lax.rsqrt(dp[:, 0:1] + dp[:, 1:2] + 1.0)
    y = dinv * xw
    yh_ref[0] = y[:, 0:HD]
    yh_ref[1] = y[:, HD:D]


_y_call = pl.pallas_call(
    _y_body,
    grid=(N // _YBLK,),
    in_specs=[
        pl.BlockSpec((_YBLK, D), lambda i: (i, 0)),
        pl.BlockSpec((D, D), lambda i: (0, 0)),
        pl.BlockSpec((_YBLK, 2), lambda i: (i, 0)),
    ],
    out_specs=pl.BlockSpec((2, _YBLK, HD), lambda i: (0, i, 0)),
    out_shape=jax.ShapeDtypeStruct((NC, N, HD), jnp.float32),
)


_CBLK = 1000


def _comb_body(acc_ref, degt_ref, out_ref):
    dp = degt_ref[...]
    dinv = lax.rsqrt(dp[:, 0:1] + dp[:, 1:2] + 1.0)
    out_ref[...] = dinv * jnp.concatenate([acc_ref[0], acc_ref[1]], axis=1)


_comb_call = pl.pallas_call(
    _comb_body,
    grid=(N // _CBLK,),
    in_specs=[
        pl.BlockSpec((2, _CBLK, HD), lambda i: (0, i, 0)),
        pl.BlockSpec((_CBLK, 2), lambda i: (i, 0)),
    ],
    out_specs=pl.BlockSpec((_CBLK, D), lambda i: (i, 0)),
    out_shape=jax.ShapeDtypeStruct((N, D), jnp.float32),
)


def kernel(edge_index, X, initial_weight, W_ih, W_hh, b_ih, b_hh):
    row32 = edge_index[0].astype(jnp.int32)
    col32 = edge_index[1].astype(jnp.int32)
    row_w = row32.reshape(NW, CH1, CHW)
    col_w = col32.reshape(NW, CH1, CHW)
    pad = EPTP - E // NS
    row_t = jnp.concatenate(
        [row32.reshape(NS, E // NS),
         jnp.zeros((NS, pad), jnp.int32)], axis=1).reshape(NS, NCH, CHS)
    col_t = jnp.concatenate(
        [col32.reshape(NS, E // NS),
         jnp.full((NS, pad), TRASH, jnp.int32)], axis=1).reshape(
             NS, NCH, CHS)

    W = _gru(initial_weight[0], W_ih, W_hh,
             b_ih.reshape(1, 3 * D), b_hh.reshape(1, 3 * D))

    deg_parts = _deg_kernel(col_w)
    degt = deg_parts.reshape(NC, DEG_N)[:, :N].T

    yh = _y_call(X, W, degt)
    acc = _prop_kernel(row_t, col_t, yh)
    out = _comb_call(acc, degt)
    return out
